# baseline (speedup 1.0000x reference)
import functools

import jax
import jax.numpy as jnp
from jax import lax
from jax.experimental import pallas as pl
from jax.experimental.pallas import tpu as pltpu

_BATCH = 16384
_EMBED = 64
_NSEM = 8


def _body(center_sm, table_any, out_vm, *sems):
    def loop(i, _):
        for u in range(_NSEM):
            j = i * _NSEM + u
            idx = center_sm[j]
            pltpu.async_copy(table_any.at[idx], out_vm.at[j], sems[u])
        return ()

    lax.fori_loop(0, _BATCH // _NSEM, loop, (), unroll=2)
    per = _BATCH // _NSEM
    for u in range(_NSEM):
        pltpu.make_async_copy(
            table_any.at[pl.ds(0, per)],
            out_vm.at[pl.ds(u * per, per)], sems[u]).wait()


def kernel(center, table):
    gather = pl.pallas_call(
        _body,
        in_specs=[
            pl.BlockSpec(memory_space=pltpu.SMEM),
            pl.BlockSpec(memory_space=pltpu.MemorySpace.HBM),
        ],
        out_specs=pl.BlockSpec(memory_space=pltpu.VMEM),
        out_shape=jax.ShapeDtypeStruct((_BATCH, _EMBED), jnp.float32),
        scratch_shapes=[pltpu.SemaphoreType.DMA] * _NSEM,
    )
    return gather(center.astype(jnp.int32), table)


# speedup vs baseline: 1.1368x; 1.1368x over previous
import functools

import jax
import jax.numpy as jnp
from jax import lax
from jax.experimental import pallas as pl
from jax.experimental.pallas import tpu as pltpu
from jax.experimental.pallas import tpu_sc as plsc

_BATCH = 16384
_EMBED = 64


def _make_gather(batch, embed):
    info = plsc.get_sparse_core_info()
    nw = info.num_cores * info.num_subcores
    b_per_w = batch // nw

    mesh = plsc.VectorSubcoreMesh(core_axis_name="c", subcore_axis_name="s")

    @functools.partial(
        pl.kernel,
        mesh=mesh,
        out_type=jax.ShapeDtypeStruct((batch, embed), jnp.float32),
        scratch_types=[
            pltpu.VMEM((b_per_w,), jnp.int32),
            pltpu.VMEM((b_per_w, embed), jnp.float32),
            pltpu.SemaphoreType.DMA,
        ],
        compiler_params=pltpu.CompilerParams(needs_layout_passes=False),
    )
    def gather(center_hbm, table_hbm, out_hbm, idx_v, out_v, sem):
        wid = lax.axis_index("s") * info.num_cores + lax.axis_index("c")
        base = wid * b_per_w
        pltpu.sync_copy(center_hbm.at[pl.ds(base, b_per_w)], idx_v)
        lanes = lax.iota(jnp.int32, 16)

        def body(g, _):
            iv = idx_v[pl.ds(g * 16, 16)]
            for j in range(16):
                sj = jnp.max(jnp.where(lanes == j, iv, 0))
                pltpu.async_copy(table_hbm.at[sj], out_v.at[g * 16 + j], sem)
            return ()

        lax.fori_loop(0, b_per_w // 16, body, ())
        pltpu.make_async_copy(table_hbm.at[pl.ds(0, b_per_w)], out_v, sem).wait()
        pltpu.sync_copy(out_v, out_hbm.at[pl.ds(base, b_per_w)])

    return gather


def kernel(center, table):
    gather = _make_gather(_BATCH, _EMBED)
    return gather(center.astype(jnp.int32), table)
